# baseline (speedup 1.0000x reference)
import functools

import jax
import jax.numpy as jnp
from jax import lax
from jax.experimental import pallas as pl
from jax.experimental.pallas import tpu as pltpu
from jax.experimental.pallas import tpu_sc as plsc

_B = 1024
_LPAD = 208
_D = 3072
_LANES = 16
_NC = 2
_NS = 16
_NW = _NC * _NS
_SEQ_PER_W = _B // _NW
_NCH = _LPAD // _LANES
_KD = _D // _LANES

_mesh = plsc.VectorSubcoreMesh(core_axis_name="c", subcore_axis_name="s")


@functools.partial(
    pl.kernel,
    mesh=_mesh,
    out_type=jax.ShapeDtypeStruct((_B, _D), jnp.float32),
    scratch_types=[
        pltpu.VMEM((_LPAD,), jnp.int32),
        pltpu.VMEM((_LPAD,), jnp.float32),
        pltpu.VMEM((_LANES, _D), jnp.float32),
        pltpu.VMEM((_D,), jnp.float32),
        pltpu.SemaphoreType.DMA,
    ],
)
def _pooled_embed(ids_hbm, w_hbm, table_hbm, out_hbm, ids_v, w_v, rows_v, acc_v, sem):
    wid = lax.axis_index("s") * _NC + lax.axis_index("c")
    base = wid * _SEQ_PER_W

    def per_seq(s, carry):
        g = base + s
        pltpu.sync_copy(ids_hbm.at[g], ids_v)
        pltpu.sync_copy(w_hbm.at[g], w_v)

        def zero_k(k, c):
            acc_v[pl.ds(k * _LANES, _LANES)] = jnp.zeros((_LANES,), jnp.float32)
            return c

        lax.fori_loop(0, _KD, zero_k, 0)

        def per_chunk(j, dsum):
            t0 = j * _LANES
            pltpu.async_copy(
                table_hbm.at[ids_v.at[pl.ds(t0, _LANES)]], rows_v, sem
            ).wait()
            wvec = w_v[pl.ds(t0, _LANES)]

            def acc_k(k, c):
                c0 = k * _LANES
                v = acc_v[pl.ds(c0, _LANES)]
                for r in range(_LANES):
                    v = v + rows_v[r, pl.ds(c0, _LANES)] * wvec[r]
                acc_v[pl.ds(c0, _LANES)] = v
                return c

            lax.fori_loop(0, _KD, acc_k, 0)
            return dsum + wvec

        dsum = lax.fori_loop(0, _NCH, per_chunk, jnp.zeros((_LANES,), jnp.float32))

        total = dsum[0]
        for r in range(1, _LANES):
            total = total + dsum[r]
        denomv = jnp.maximum(jnp.full((_LANES,), total, jnp.float32), 1e-6)
        rv = 1.0 / denomv

        def scale_k(k, c):
            c0 = k * _LANES
            acc_v[pl.ds(c0, _LANES)] = acc_v[pl.ds(c0, _LANES)] * rv
            return c

        lax.fori_loop(0, _KD, scale_k, 0)
        pltpu.sync_copy(acc_v, out_hbm.at[g])
        return carry

    lax.fori_loop(0, _SEQ_PER_W, per_seq, 0)


def kernel(input_ids, attention_mask, table):
    ids = jnp.asarray(input_ids, jnp.int32)
    ids = jnp.clip(ids, 0, table.shape[0] - 1)
    w = attention_mask.astype(jnp.float32)
    pad = _LPAD - ids.shape[1]
    ids = jnp.pad(ids, ((0, 0), (0, pad)))
    w = jnp.pad(w, ((0, 0), (0, pad)))
    return _pooled_embed(ids, w, table)


# speedup vs baseline: 2.4801x; 2.4801x over previous
import functools

import jax
import jax.numpy as jnp
from jax import lax
from jax.experimental import pallas as pl
from jax.experimental.pallas import tpu as pltpu
from jax.experimental.pallas import tpu_sc as plsc

_B = 1024
_LPAD = 208
_D = 3072
_LANES = 16
_NC = 2
_NS = 16
_NW = _NC * _NS
_SEQ_PER_W = _B // _NW
_NCH = _LPAD // _LANES
_KD = _D // _LANES
_IDSPAN = 131072

_mesh = plsc.VectorSubcoreMesh(core_axis_name="c", subcore_axis_name="s")


@functools.partial(
    pl.kernel,
    mesh=_mesh,
    out_type=jax.ShapeDtypeStruct((_B, _D), jnp.float32),
    scratch_types=[
        pltpu.VMEM((_SEQ_PER_W * _LPAD,), jnp.int32),
        pltpu.VMEM((_SEQ_PER_W * _LPAD,), jnp.float32),
        pltpu.VMEM((_LANES, _D), jnp.float32),
        pltpu.VMEM((_LANES, _D), jnp.float32),
        pltpu.VMEM((_D,), jnp.float32),
        pltpu.SemaphoreType.DMA,
        pltpu.SemaphoreType.DMA,
    ],
)
def _pooled_embed(
    ids_hbm, w_hbm, table_hbm, out_hbm,
    ids_v, w_v, rows_a, rows_b, acc_v,
    sem_a, sem_b,
):
    wid = lax.axis_index("s") * _NC + lax.axis_index("c")
    base = wid * _SEQ_PER_W

    def gather(o, j, rows, sem):
        pltpu.async_copy(
            table_hbm.at[ids_v.at[pl.ds(o + j * _LANES, _LANES)]], rows, sem
        )

    def gather_wait(rows, sem):
        pltpu.make_async_copy(
            table_hbm.at[ids_v.at[pl.ds(0, _LANES)]], rows, sem
        ).wait()

    def accumulate(rows, wvec):
        ws = [wvec[r] for r in range(_LANES)]

        def acc_k(k, c):
            c0 = k * _LANES
            v = acc_v[pl.ds(c0, _LANES)]
            for r in range(_LANES):
                v = v + rows[r, pl.ds(c0, _LANES)] * ws[r]
            acc_v[pl.ds(c0, _LANES)] = v
            return c

        lax.fori_loop(0, _KD, acc_k, 0)

    pltpu.sync_copy(
        ids_hbm.at[pl.ds(base * _LPAD, _SEQ_PER_W * _LPAD)], ids_v
    )
    pltpu.sync_copy(
        w_hbm.at[pl.ds(base * _LPAD, _SEQ_PER_W * _LPAD)], w_v
    )

    def per_seq(s, carry):
        g = base + s
        o = s * _LPAD

        dsum = jnp.zeros((_LANES,), jnp.float32)
        for j in range(_NCH):
            dsum = dsum + w_v[pl.ds(o + j * _LANES, _LANES)]
        total = dsum[0]
        for r in range(1, _LANES):
            total = total + dsum[r]

        def zero_k(k, c):
            acc_v[pl.ds(k * _LANES, _LANES)] = jnp.zeros((_LANES,), jnp.float32)
            return c

        lax.fori_loop(0, _KD, zero_k, 0)

        @pl.when(total > 0.0)
        def _():
            gather(o, 0, rows_a, sem_a)

        for j in range(_NCH):
            cur, csem = (rows_a, sem_a) if j % 2 == 0 else (rows_b, sem_b)
            any_active = total > jnp.float32(j * _LANES)
            if j + 1 < _NCH:
                nrows, nsem = (rows_a, sem_a) if (j + 1) % 2 == 0 else (rows_b, sem_b)

                @pl.when(total > jnp.float32((j + 1) * _LANES))
                def _(nrows=nrows, nsem=nsem, j=j):
                    gather(o, j + 1, nrows, nsem)

            @pl.when(any_active)
            def _(cur=cur, csem=csem, j=j):
                gather_wait(cur, csem)
                accumulate(cur, w_v[pl.ds(o + j * _LANES, _LANES)])

        denom = jnp.maximum(total, 1e-6)
        rv = 1.0 / jnp.full((_LANES,), denom, jnp.float32)

        def scale_k(k, c):
            c0 = k * _LANES
            acc_v[pl.ds(c0, _LANES)] = acc_v[pl.ds(c0, _LANES)] * rv
            return c

        lax.fori_loop(0, _KD, scale_k, 0)
        pltpu.sync_copy(acc_v, out_hbm.at[g])
        return carry

    lax.fori_loop(0, _SEQ_PER_W, per_seq, 0)


def kernel(input_ids, attention_mask, table):
    vocab = table.shape[0]
    ids = jnp.clip(jnp.asarray(input_ids, jnp.int32), 0, vocab - 1)
    active = attention_mask != 0
    key = jnp.where(active, ids, ids + _IDSPAN)
    pad = _LPAD - key.shape[1]
    key = jnp.pad(key, ((0, 0), (0, pad)), constant_values=2 * _IDSPAN)
    key = jnp.sort(key, axis=1)
    ids_sorted = (key % _IDSPAN).reshape(-1)
    w_sorted = (key < _IDSPAN).astype(jnp.float32).reshape(-1)
    return _pooled_embed(ids_sorted, w_sorted, table)
